# speedup vs baseline: 7.0085x; 7.0085x over previous
import functools
import jax
import jax.numpy as jnp
from jax import lax
from jax.experimental import pallas as pl
from jax.experimental.pallas import tpu as pltpu
from jax.experimental.pallas import tpu_sc as plsc

N = 50000
E = 1600000
B = 16
NSTEP = 4
DT = 1.0 / NSTEP
DT6 = DT / 6.0

NT = 16
NP = 50176
RANGE = NP // NT
NCH = 64
NCHN = RANGE // NCH

EC = 256
EP = 1601536
EPT = EP // NT
ECN = EPT // EC

_mesh = plsc.VectorSubcoreMesh(core_axis_name="c", subcore_axis_name="s",
                               num_cores=1)


def _sc_body(x0t, src_r, dst_r, w_r, kd_r, kr_r,
             xb, accb, wrow,
             xe_sh, agg_sh,
             rows, src_b, dst_b, w_b,
             agg_b, xe_b, wr_b, acc_b, xb_b, kd_b, kr_b, z_b,
             sem):
    sid = lax.axis_index("s")
    nbase = sid * RANGE
    rbase = sid * (EPT // 128)
    wbase = sid * (EPT // 16)
    cbase = sid * (RANGE // 16)

    def fill(buf, val):
        for j in range(NCH):
            buf[j] = jnp.zeros((B,), jnp.float32) + val

    def edge_pass(_t, carry):
        def chunk(q, c):
            rb = rbase + q * 2
            wb = wbase + q * 16
            pltpu.sync_copy(src_r.at[pl.ds(rb, 2)], src_b)
            pltpu.sync_copy(dst_r.at[pl.ds(rb, 2)], dst_b)
            pltpu.sync_copy(w_r.at[pl.ds(wb, 16)], w_b)
            pltpu.async_copy(xe_sh.at[src_b.at[0]], rows.at[pl.ds(0, 128)],
                             sem).wait()
            pltpu.async_copy(xe_sh.at[src_b.at[1]], rows.at[pl.ds(128, 128)],
                             sem).wait()
            for g in range(16):
                wv = w_b[g]
                for l in range(16):
                    j = g * 16 + l
                    rows[j] = rows[j] * wv[l]
            pltpu.sync_copy(rows.at[pl.ds(0, 128)],
                            agg_sh.at[dst_b.at[0]], add=True)
            pltpu.sync_copy(rows.at[pl.ds(128, 128)],
                            agg_sh.at[dst_b.at[1]], add=True)
            return c
        return lax.fori_loop(0, ECN, chunk, carry)

    fill(z_b, 1.0)
    def p_ones(m, c):
        pltpu.sync_copy(z_b, xe_sh.at[pl.ds(nbase + m * NCH, NCH)])
        return c
    lax.fori_loop(0, NCHN, p_ones, 0)
    fill(z_b, 0.0)
    def p_zero(m, c):
        nb = nbase + m * NCH
        pltpu.sync_copy(z_b, agg_sh.at[pl.ds(nb, NCH)])
        pltpu.sync_copy(z_b, accb.at[pl.ds(nb, NCH)])
        return c
    lax.fori_loop(0, NCHN, p_zero, 0)
    plsc.subcore_barrier()

    edge_pass(0, 0)
    plsc.subcore_barrier()

    def p_w(m, c):
        nb = nbase + m * NCH
        cb = cbase + m * (NCH // 16)
        pltpu.sync_copy(agg_sh.at[pl.ds(nb, NCH)], agg_b)
        pltpu.sync_copy(kd_r.at[pl.ds(cb, NCH // 16)], kd_b)
        pltpu.sync_copy(kr_r.at[pl.ds(cb, NCH // 16)], kr_b)
        for g in range(NCH // 16):
            kdv = kd_b[g]
            krv = kr_b[g]
            for l in range(16):
                j = g * 16 + l
                wr_b[j] = agg_b[j] * kdv[l] - krv[l]
        pltpu.sync_copy(wr_b, wrow.at[pl.ds(nb, NCH)])
        pltpu.sync_copy(x0t.at[pl.ds(nb, NCH)], xe_b)
        pltpu.sync_copy(xe_b, xe_sh.at[pl.ds(nb, NCH)])
        pltpu.sync_copy(xe_b, xb.at[pl.ds(nb, NCH)])
        pltpu.sync_copy(z_b, agg_sh.at[pl.ds(nb, NCH)])
        return c
    lax.fori_loop(0, NCHN, p_w, 0)
    plsc.subcore_barrier()

    def eval_body(t, carry):
        i = lax.rem(t, 4)
        is3 = (i == 3)
        wgt = jnp.where((i == 1) | (i == 2), 2.0, 1.0).astype(jnp.float32)
        ci = jnp.where(i == 2, DT, 0.5 * DT).astype(jnp.float32)

        edge_pass(t, carry)
        plsc.subcore_barrier()

        def node_chunk(m, c):
            nb = nbase + m * NCH
            cb = cbase + m * (NCH // 16)
            pltpu.sync_copy(agg_sh.at[pl.ds(nb, NCH)], agg_b)
            pltpu.sync_copy(xe_sh.at[pl.ds(nb, NCH)], xe_b)
            pltpu.sync_copy(wrow.at[pl.ds(nb, NCH)], wr_b)
            pltpu.sync_copy(accb.at[pl.ds(nb, NCH)], acc_b)
            pltpu.sync_copy(xb.at[pl.ds(nb, NCH)], xb_b)
            pltpu.sync_copy(kd_r.at[pl.ds(cb, NCH // 16)], kd_b)
            pltpu.sync_copy(kr_r.at[pl.ds(cb, NCH // 16)], kr_b)
            for g in range(NCH // 16):
                kdv = kd_b[g]
                krv = kr_b[g]
                for l in range(16):
                    j = g * 16 + l
                    xe = xe_b[j]
                    deriv = (agg_b[j] * kdv[l] - wr_b[j] * xe
                             - (xe * xe) * krv[l])
                    acc_new = acc_b[j] + deriv * wgt
                    xbv = xb_b[j]
                    xb_new = jnp.where(is3, xbv + acc_new * DT6, xbv)
                    xe_new = jnp.where(is3, xb_new, xbv + deriv * ci)
                    acc_b[j] = jnp.where(is3, jnp.zeros((B,), jnp.float32),
                                         acc_new)
                    xb_b[j] = xb_new
                    xe_b[j] = xe_new
            pltpu.sync_copy(xe_b, xe_sh.at[pl.ds(nb, NCH)])
            pltpu.sync_copy(acc_b, accb.at[pl.ds(nb, NCH)])
            pltpu.sync_copy(xb_b, xb.at[pl.ds(nb, NCH)])
            pltpu.sync_copy(z_b, agg_sh.at[pl.ds(nb, NCH)])
            return c
        lax.fori_loop(0, NCHN, node_chunk, carry)
        plsc.subcore_barrier()
        return carry

    lax.fori_loop(0, NSTEP * 4, eval_body, 0)


@functools.partial(
    pl.kernel,
    out_type=[jax.ShapeDtypeStruct((NP, B), jnp.float32),
              jax.ShapeDtypeStruct((NP, B), jnp.float32),
              jax.ShapeDtypeStruct((NP, B), jnp.float32)],
    mesh=_mesh,
    compiler_params=pltpu.CompilerParams(use_tc_tiling_on_sc=False),
    scratch_types=[
        pltpu.VMEM_SHARED((NP, B), jnp.float32),
        pltpu.VMEM_SHARED((NP, B), jnp.float32),
        pltpu.VMEM((EC, B), jnp.float32),
        pltpu.VMEM((2, 128), jnp.int32),
        pltpu.VMEM((2, 128), jnp.int32),
        pltpu.VMEM((16, 16), jnp.float32),
        pltpu.VMEM((NCH, B), jnp.float32),
        pltpu.VMEM((NCH, B), jnp.float32),
        pltpu.VMEM((NCH, B), jnp.float32),
        pltpu.VMEM((NCH, B), jnp.float32),
        pltpu.VMEM((NCH, B), jnp.float32),
        pltpu.VMEM((NCH // 16, 16), jnp.float32),
        pltpu.VMEM((NCH // 16, 16), jnp.float32),
        pltpu.VMEM((NCH, B), jnp.float32),
        pltpu.SemaphoreType.DMA,
    ])
def _rd_ode_sc(x0t, src_r, dst_r, w_r, kd_r, kr_r, *rest):
    _sc_body(x0t, src_r, dst_r, w_r, kd_r, kr_r, *rest)


@jax.jit
def kernel(inputs, ind, edge_index, edge_w, k_diff, k_reac):
    del ind
    x0 = inputs[:, 0, :, -1]
    x0t = jnp.zeros((NP, B), jnp.float32).at[:N].set(x0.T)

    pad_e = EP - E
    src = jnp.concatenate([edge_index[0].astype(jnp.int32),
                           jnp.zeros((pad_e,), jnp.int32)])
    dst = jnp.concatenate([edge_index[1].astype(jnp.int32),
                           jnp.full((pad_e,), NP - 1, jnp.int32)])
    w = jnp.concatenate([edge_w, jnp.zeros((pad_e,), jnp.float32)])
    src_r = src.reshape(EP // 128, 128)
    dst_r = dst.reshape(EP // 128, 128)
    w_r = w.reshape(EP // 16, 16)

    kd_r = jnp.zeros((NP,), jnp.float32).at[:N].set(k_diff).reshape(NP // 16, 16)
    kr_r = jnp.zeros((NP,), jnp.float32).at[:N].set(k_reac).reshape(NP // 16, 16)

    xb, _accb, _wrow = _rd_ode_sc(x0t, src_r, dst_r, w_r, kd_r, kr_r)
    return xb[:N].T[None]


# speedup vs baseline: 11.5007x; 1.6410x over previous
import functools
import jax
import jax.numpy as jnp
from jax import lax
from jax.experimental import pallas as pl
from jax.experimental.pallas import tpu as pltpu
from jax.experimental.pallas import tpu_sc as plsc

N = 50000
E = 1600000
B = 16
NSTEP = 4
DT = 1.0 / NSTEP
DT6 = DT / 6.0

NT = 16
NP = 50176
RANGE = NP // NT
NCH = 64
NCHN = RANGE // NCH

EC = 128
EP = 1605632
EPT = EP // NT
ECN = EPT // EC

_mesh = plsc.VectorSubcoreMesh(core_axis_name="c", subcore_axis_name="s",
                               num_cores=1)


def _sc_body(x0t, edata, wdata, kd_r, kr_r,
             xb, accb, wrow,
             xe_sh, agg_sh,
             rows0, rows1, ed0, ed1, wd0, wd1, si0, si1,
             agg0, xe0, wr0, acc0, xb0, kd0, kr0, z_b,
             sem_e0, sem_e1, sem_g0, sem_g1, sem_s0, sem_s1,
             sem_n0):
    sid = lax.axis_index("s")
    nbase = sid * RANGE
    cbase = sid * (RANGE // 16)
    cid0 = sid * ECN

    rows = (rows0, rows1)
    ed = (ed0, ed1)
    wd = (wd0, wd1)
    si = (si0, si1)
    agg_b, xe_b, wr_b, acc_b, xb_b, kd_b, kr_b = (
        agg0, xe0, wr0, acc0, xb0, kd0, kr0)
    sem_e = (sem_e0, sem_e1)
    sem_g = (sem_g0, sem_g1)
    sem_s = (sem_s0, sem_s1)
    sem_n = sem_n0

    def ed_fetch(q, p):
        return (pltpu.make_async_copy(edata.at[pl.ds((cid0 + q) * 2, 2)],
                                      ed[p], sem_e[p]),
                pltpu.make_async_copy(wdata.at[pl.ds((cid0 + q) * 8, 8)],
                                      wd[p], sem_e[p]))

    def gathers(p):
        return (pltpu.make_async_copy(xe_sh.at[ed[p].at[0]],
                                      rows[p], sem_g[p]),)

    def scatters(p):
        return (pltpu.make_async_copy(rows[p],
                                      agg_sh.at[si[p].at[0]], sem_s[p]),)

    def scale(p):
        r = rows[p]
        e = ed[p]
        s = si[p]
        for g in range(8):
            wv = wd[p][g]
            for l in range(16):
                j = g * 16 + l
                r[j] = r[j] * wv[l]
        for k in range(8):
            s[0, pl.ds(k * 16, 16)] = e[1, pl.ds(k * 16, 16)]

    def edge_pass(carry):
        for d in ed_fetch(0, 0):
            d.start()
        for d in ed_fetch(0, 0):
            d.wait()
        for g in gathers(0):
            g.start()
        for d in ed_fetch(1, 1):
            d.start()

        def q2_body(q2, c):
            last = q2 >= ECN // 2 - 1
            for p in (0, 1):
                q = q2 * 2 + p
                if p == 0:
                    for d in ed_fetch(0, 1):
                        d.wait()
                else:
                    @pl.when(jnp.logical_not(last))
                    def _():
                        for d in ed_fetch(0, 0):
                            d.wait()
                for g in gathers(p):
                    g.wait()
                if p == 0:
                    @pl.when(q2 > 0)
                    def _():
                        for s in scatters(1):
                            s.wait()
                else:
                    for s in scatters(0):
                        s.wait()
                if p == 0:
                    for g in gathers(1):
                        g.start()
                else:
                    @pl.when(jnp.logical_not(last))
                    def _():
                        for g in gathers(0):
                            g.start()
                scale(p)
                for s in scatters(p):
                    s.start(add=True)
                @pl.when(jnp.logical_not(last))
                def _():
                    for d in ed_fetch(q + 2, p):
                        d.start()
            return c

        c = lax.fori_loop(0, ECN // 2, q2_body, carry)
        for s in scatters(1):
            s.wait()
        return c

    def node_io(m):
        nb = nbase + m * NCH
        cb = cbase + m * (NCH // 16)
        return (pltpu.make_async_copy(agg_sh.at[pl.ds(nb, NCH)],
                                      agg_b, sem_n),
                pltpu.make_async_copy(xe_sh.at[pl.ds(nb, NCH)],
                                      xe_b, sem_n),
                pltpu.make_async_copy(wrow.at[pl.ds(nb, NCH)],
                                      wr_b, sem_n),
                pltpu.make_async_copy(accb.at[pl.ds(nb, NCH)],
                                      acc_b, sem_n),
                pltpu.make_async_copy(xb.at[pl.ds(nb, NCH)],
                                      xb_b, sem_n),
                pltpu.make_async_copy(kd_r.at[pl.ds(cb, NCH // 16)],
                                      kd_b, sem_n),
                pltpu.make_async_copy(kr_r.at[pl.ds(cb, NCH // 16)],
                                      kr_b, sem_n))

    def node_compute(wgt, bz, ciz, az):
        for g in range(NCH // 16):
            kdv = kd_b[g]
            krv = kr_b[g]
            for l in range(16):
                j = g * 16 + l
                xe = xe_b[j]
                deriv = (agg_b[j] * kdv[l] - wr_b[j] * xe
                         - (xe * xe) * krv[l])
                acc_new = acc_b[j] + deriv * wgt
                t = acc_new * bz
                xb_new = xb_b[j] + t
                xe_b[j] = xb_new + deriv * ciz
                xb_b[j] = xb_new
                acc_b[j] = acc_new * az

    for j in range(NCH):
        z_b[j] = jnp.zeros((B,), jnp.float32) + 1.0
    def p_ones(m, c):
        pltpu.sync_copy(z_b, xe_sh.at[pl.ds(nbase + m * NCH, NCH)])
        return c
    lax.fori_loop(0, NCHN, p_ones, 0)
    for j in range(NCH):
        z_b[j] = jnp.zeros((B,), jnp.float32)
    def p_zero(m, c):
        nb = nbase + m * NCH
        pltpu.sync_copy(z_b, agg_sh.at[pl.ds(nb, NCH)])
        pltpu.sync_copy(z_b, accb.at[pl.ds(nb, NCH)])
        return c
    lax.fori_loop(0, NCHN, p_zero, 0)
    plsc.subcore_barrier()

    edge_pass(0)
    plsc.subcore_barrier()

    def p_w(m, c):
        nb = nbase + m * NCH
        cb = cbase + m * (NCH // 16)
        pltpu.sync_copy(agg_sh.at[pl.ds(nb, NCH)], agg_b)
        pltpu.sync_copy(kd_r.at[pl.ds(cb, NCH // 16)], kd_b)
        pltpu.sync_copy(kr_r.at[pl.ds(cb, NCH // 16)], kr_b)
        for g in range(NCH // 16):
            kdv = kd_b[g]
            krv = kr_b[g]
            for l in range(16):
                j = g * 16 + l
                wr_b[j] = agg_b[j] * kdv[l] - krv[l]
        pltpu.sync_copy(wr_b, wrow.at[pl.ds(nb, NCH)])
        pltpu.sync_copy(x0t.at[pl.ds(nb, NCH)], xe_b)
        pltpu.sync_copy(xe_b, xe_sh.at[pl.ds(nb, NCH)])
        pltpu.sync_copy(xe_b, xb.at[pl.ds(nb, NCH)])
        pltpu.sync_copy(z_b, agg_sh.at[pl.ds(nb, NCH)])
        return c
    lax.fori_loop(0, NCHN, p_w, 0)
    plsc.subcore_barrier()

    def eval_body(t, carry):
        i = lax.rem(t, 4)
        is3 = i == 3
        wgt = jnp.where((i == 1) | (i == 2), 2.0, 1.0).astype(jnp.float32)
        bz = jnp.where(is3, DT6, 0.0).astype(jnp.float32)
        ciz = jnp.where(is3, 0.0,
                        jnp.where(i == 2, DT, 0.5 * DT)).astype(jnp.float32)
        az = jnp.where(is3, 0.0, 1.0).astype(jnp.float32)

        edge_pass(carry)
        plsc.subcore_barrier()

        def node_chunk(m, c):
            nb = nbase + m * NCH
            cb = cbase + m * (NCH // 16)
            pltpu.sync_copy(agg_sh.at[pl.ds(nb, NCH)], agg_b)
            pltpu.sync_copy(xe_sh.at[pl.ds(nb, NCH)], xe_b)
            pltpu.sync_copy(wrow.at[pl.ds(nb, NCH)], wr_b)
            pltpu.sync_copy(accb.at[pl.ds(nb, NCH)], acc_b)
            pltpu.sync_copy(xb.at[pl.ds(nb, NCH)], xb_b)
            pltpu.sync_copy(kd_r.at[pl.ds(cb, NCH // 16)], kd_b)
            pltpu.sync_copy(kr_r.at[pl.ds(cb, NCH // 16)], kr_b)
            node_compute(wgt, bz, ciz, az)
            pltpu.sync_copy(xe_b, xe_sh.at[pl.ds(nb, NCH)])
            pltpu.sync_copy(acc_b, accb.at[pl.ds(nb, NCH)])
            pltpu.sync_copy(xb_b, xb.at[pl.ds(nb, NCH)])
            pltpu.sync_copy(z_b, agg_sh.at[pl.ds(nb, NCH)])
            return c
        lax.fori_loop(0, NCHN, node_chunk, carry)
        plsc.subcore_barrier()
        return carry

    lax.fori_loop(0, NSTEP * 4, eval_body, 0)


@functools.partial(
    pl.kernel,
    out_type=[jax.ShapeDtypeStruct((NP, B), jnp.float32),
              jax.ShapeDtypeStruct((NP, B), jnp.float32),
              jax.ShapeDtypeStruct((NP, B), jnp.float32)],
    mesh=_mesh,
    compiler_params=pltpu.CompilerParams(use_tc_tiling_on_sc=False),
    scratch_types=[
        pltpu.VMEM_SHARED((NP, B), jnp.float32),
        pltpu.VMEM_SHARED((NP, B), jnp.float32),
        pltpu.VMEM((EC, B), jnp.float32),
        pltpu.VMEM((EC, B), jnp.float32),
        pltpu.VMEM((2, 128), jnp.int32),
        pltpu.VMEM((2, 128), jnp.int32),
        pltpu.VMEM((8, 16), jnp.float32),
        pltpu.VMEM((8, 16), jnp.float32),
        pltpu.VMEM((1, 128), jnp.int32),
        pltpu.VMEM((1, 128), jnp.int32),
        pltpu.VMEM((NCH, B), jnp.float32),
        pltpu.VMEM((NCH, B), jnp.float32),
        pltpu.VMEM((NCH, B), jnp.float32),
        pltpu.VMEM((NCH, B), jnp.float32),
        pltpu.VMEM((NCH, B), jnp.float32),
        pltpu.VMEM((NCH // 16, 16), jnp.float32),
        pltpu.VMEM((NCH // 16, 16), jnp.float32),
        pltpu.VMEM((NCH, B), jnp.float32),
        pltpu.SemaphoreType.DMA,
        pltpu.SemaphoreType.DMA,
        pltpu.SemaphoreType.DMA,
        pltpu.SemaphoreType.DMA,
        pltpu.SemaphoreType.DMA,
        pltpu.SemaphoreType.DMA,
        pltpu.SemaphoreType.DMA,
    ])
def _rd_ode_sc(x0t, edata, wdata, kd_r, kr_r, *rest):
    _sc_body(x0t, edata, wdata, kd_r, kr_r, *rest)


@jax.jit
def kernel(inputs, ind, edge_index, edge_w, k_diff, k_reac):
    del ind
    x0 = inputs[:, 0, :, -1]
    x0t = jnp.zeros((NP, B), jnp.float32).at[:N].set(x0.T)

    pad_e = EP - E
    src = jnp.concatenate([edge_index[0].astype(jnp.int32),
                           jnp.zeros((pad_e,), jnp.int32)])
    dst = jnp.concatenate([edge_index[1].astype(jnp.int32),
                           jnp.full((pad_e,), NP - 1, jnp.int32)])
    w = jnp.concatenate([edge_w, jnp.zeros((pad_e,), jnp.float32)])
    nchunks = EP // EC
    edata = jnp.concatenate([src.reshape(nchunks, 1, 128),
                             dst.reshape(nchunks, 1, 128)],
                            axis=1).reshape(nchunks * 2, 128)
    wdata = w.reshape(nchunks * 8, 16)

    kd_r = jnp.zeros((NP,), jnp.float32).at[:N].set(k_diff).reshape(NP // 16, 16)
    kr_r = jnp.zeros((NP,), jnp.float32).at[:N].set(k_reac).reshape(NP // 16, 16)

    xb, _accb, _wrow = _rd_ode_sc(x0t, edata, wdata, kd_r, kr_r)
    return xb[:N].T[None]


# speedup vs baseline: 16.1234x; 1.4020x over previous
import functools
import jax
import jax.numpy as jnp
from jax import lax
from jax.experimental import pallas as pl
from jax.experimental.pallas import tpu as pltpu
from jax.experimental.pallas import tpu_sc as plsc

N = 50000
E = 1600000
B = 16
NSTEP = 4
DT = 1.0 / NSTEP
DT6 = DT / 6.0

NT = 16
NP = 50176
RANGE = NP // NT
NCH = 64
NCHN = RANGE // NCH

EC = 256
EP = 1605632
EPT = EP // NT
ECN = EPT // EC

_mesh = plsc.VectorSubcoreMesh(core_axis_name="c", subcore_axis_name="s",
                               num_cores=1)


def _sc_body(x0t, edata, wdata, kd_r, kr_r,
             xb, accb, wrow,
             xe_sh, agg_sh,
             rows0, rows1, ed0, ed1, wd0, wd1, si0, si1,
             agg0, xe0, wr0, acc0, xb0, kd0, kr0, z_b,
             sem_e0, sem_e1, sem_g0, sem_g1, sem_s0, sem_s1,
             sem_n0):
    sid = lax.axis_index("s")
    nbase = sid * RANGE
    cbase = sid * (RANGE // 16)
    cid0 = sid * ECN

    rows = (rows0, rows1)
    ed = (ed0, ed1)
    wd = (wd0, wd1)
    si = (si0, si1)
    agg_b, xe_b, wr_b, acc_b, xb_b, kd_b, kr_b = (
        agg0, xe0, wr0, acc0, xb0, kd0, kr0)
    sem_e = (sem_e0, sem_e1)
    sem_g = (sem_g0, sem_g1)
    sem_s = (sem_s0, sem_s1)
    sem_n = sem_n0

    def ed_fetch(q, p):
        return (pltpu.make_async_copy(edata.at[pl.ds((cid0 + q) * 4, 4)],
                                      ed[p], sem_e[p]),
                pltpu.make_async_copy(wdata.at[pl.ds((cid0 + q) * 16, 16)],
                                      wd[p], sem_e[p]))

    def gathers(p):
        return (pltpu.make_async_copy(xe_sh.at[ed[p].at[0]],
                                      rows[p].at[pl.ds(0, 128)], sem_g[p]),
                pltpu.make_async_copy(xe_sh.at[ed[p].at[1]],
                                      rows[p].at[pl.ds(128, 128)], sem_g[p]))

    def scatters(p):
        return (pltpu.make_async_copy(rows[p].at[pl.ds(0, 128)],
                                      agg_sh.at[si[p].at[0]], sem_s[p]),
                pltpu.make_async_copy(rows[p].at[pl.ds(128, 128)],
                                      agg_sh.at[si[p].at[1]], sem_s[p]))

    def scale(p):
        r = rows[p]
        e = ed[p]
        s = si[p]
        for g in range(16):
            wv = wd[p][g]
            for l in range(16):
                j = g * 16 + l
                r[j] = r[j] * wv[l]
        for rr in range(2):
            for k in range(8):
                s[rr, pl.ds(k * 16, 16)] = e[2 + rr, pl.ds(k * 16, 16)]

    def edge_pass(carry):
        for d in ed_fetch(0, 0):
            d.start()
        for d in ed_fetch(0, 0):
            d.wait()
        for g in gathers(0):
            g.start()
        for d in ed_fetch(1, 1):
            d.start()

        def q2_body(q2, c):
            last = q2 >= ECN // 2 - 1
            for p in (0, 1):
                q = q2 * 2 + p
                if p == 0:
                    for d in ed_fetch(0, 1):
                        d.wait()
                else:
                    @pl.when(jnp.logical_not(last))
                    def _():
                        for d in ed_fetch(0, 0):
                            d.wait()
                for g in gathers(p):
                    g.wait()
                if p == 0:
                    @pl.when(q2 > 0)
                    def _():
                        for s in scatters(1):
                            s.wait()
                else:
                    for s in scatters(0):
                        s.wait()
                if p == 0:
                    for g in gathers(1):
                        g.start()
                else:
                    @pl.when(jnp.logical_not(last))
                    def _():
                        for g in gathers(0):
                            g.start()
                scale(p)
                for s in scatters(p):
                    s.start(add=True)
                @pl.when(jnp.logical_not(last))
                def _():
                    for d in ed_fetch(q + 2, p):
                        d.start()
            return c

        c = lax.fori_loop(0, ECN // 2, q2_body, carry)
        for s in scatters(1):
            s.wait()
        return c

    def node_io(m):
        nb = nbase + m * NCH
        cb = cbase + m * (NCH // 16)
        return (pltpu.make_async_copy(agg_sh.at[pl.ds(nb, NCH)],
                                      agg_b, sem_n),
                pltpu.make_async_copy(xe_sh.at[pl.ds(nb, NCH)],
                                      xe_b, sem_n),
                pltpu.make_async_copy(wrow.at[pl.ds(nb, NCH)],
                                      wr_b, sem_n),
                pltpu.make_async_copy(accb.at[pl.ds(nb, NCH)],
                                      acc_b, sem_n),
                pltpu.make_async_copy(xb.at[pl.ds(nb, NCH)],
                                      xb_b, sem_n),
                pltpu.make_async_copy(kd_r.at[pl.ds(cb, NCH // 16)],
                                      kd_b, sem_n),
                pltpu.make_async_copy(kr_r.at[pl.ds(cb, NCH // 16)],
                                      kr_b, sem_n))

    def node_compute(wgt, bz, ciz, az):
        for g in range(NCH // 16):
            kdv = kd_b[g]
            krv = kr_b[g]
            for l in range(16):
                j = g * 16 + l
                xe = xe_b[j]
                deriv = (agg_b[j] * kdv[l] - wr_b[j] * xe
                         - (xe * xe) * krv[l])
                acc_new = acc_b[j] + deriv * wgt
                t = acc_new * bz
                xb_new = xb_b[j] + t
                xe_b[j] = xb_new + deriv * ciz
                xb_b[j] = xb_new
                acc_b[j] = acc_new * az

    for j in range(NCH):
        z_b[j] = jnp.zeros((B,), jnp.float32) + 1.0
    def p_ones(m, c):
        pltpu.sync_copy(z_b, xe_sh.at[pl.ds(nbase + m * NCH, NCH)])
        return c
    lax.fori_loop(0, NCHN, p_ones, 0)
    for j in range(NCH):
        z_b[j] = jnp.zeros((B,), jnp.float32)
    def p_zero(m, c):
        nb = nbase + m * NCH
        pltpu.sync_copy(z_b, agg_sh.at[pl.ds(nb, NCH)])
        pltpu.sync_copy(z_b, accb.at[pl.ds(nb, NCH)])
        return c
    lax.fori_loop(0, NCHN, p_zero, 0)
    plsc.subcore_barrier()

    edge_pass(0)
    plsc.subcore_barrier()

    def p_w(m, c):
        nb = nbase + m * NCH
        cb = cbase + m * (NCH // 16)
        pltpu.sync_copy(agg_sh.at[pl.ds(nb, NCH)], agg_b)
        pltpu.sync_copy(kd_r.at[pl.ds(cb, NCH // 16)], kd_b)
        pltpu.sync_copy(kr_r.at[pl.ds(cb, NCH // 16)], kr_b)
        for g in range(NCH // 16):
            kdv = kd_b[g]
            krv = kr_b[g]
            for l in range(16):
                j = g * 16 + l
                wr_b[j] = agg_b[j] * kdv[l] - krv[l]
        pltpu.sync_copy(wr_b, wrow.at[pl.ds(nb, NCH)])
        pltpu.sync_copy(x0t.at[pl.ds(nb, NCH)], xe_b)
        pltpu.sync_copy(xe_b, xe_sh.at[pl.ds(nb, NCH)])
        pltpu.sync_copy(xe_b, xb.at[pl.ds(nb, NCH)])
        pltpu.sync_copy(z_b, agg_sh.at[pl.ds(nb, NCH)])
        return c
    lax.fori_loop(0, NCHN, p_w, 0)
    plsc.subcore_barrier()

    def eval_body(t, carry):
        i = lax.rem(t, 4)
        is3 = i == 3
        wgt = jnp.where((i == 1) | (i == 2), 2.0, 1.0).astype(jnp.float32)
        bz = jnp.where(is3, DT6, 0.0).astype(jnp.float32)
        ciz = jnp.where(is3, 0.0,
                        jnp.where(i == 2, DT, 0.5 * DT)).astype(jnp.float32)
        az = jnp.where(is3, 0.0, 1.0).astype(jnp.float32)

        edge_pass(carry)
        plsc.subcore_barrier()

        def node_chunk(m, c):
            nb = nbase + m * NCH
            cb = cbase + m * (NCH // 16)
            pltpu.sync_copy(agg_sh.at[pl.ds(nb, NCH)], agg_b)
            pltpu.sync_copy(xe_sh.at[pl.ds(nb, NCH)], xe_b)
            pltpu.sync_copy(wrow.at[pl.ds(nb, NCH)], wr_b)
            pltpu.sync_copy(accb.at[pl.ds(nb, NCH)], acc_b)
            pltpu.sync_copy(xb.at[pl.ds(nb, NCH)], xb_b)
            pltpu.sync_copy(kd_r.at[pl.ds(cb, NCH // 16)], kd_b)
            pltpu.sync_copy(kr_r.at[pl.ds(cb, NCH // 16)], kr_b)
            node_compute(wgt, bz, ciz, az)
            pltpu.sync_copy(xe_b, xe_sh.at[pl.ds(nb, NCH)])
            pltpu.sync_copy(acc_b, accb.at[pl.ds(nb, NCH)])
            pltpu.sync_copy(xb_b, xb.at[pl.ds(nb, NCH)])
            pltpu.sync_copy(z_b, agg_sh.at[pl.ds(nb, NCH)])
            return c
        lax.fori_loop(0, NCHN, node_chunk, carry)
        plsc.subcore_barrier()
        return carry

    lax.fori_loop(0, NSTEP * 4, eval_body, 0)


@functools.partial(
    pl.kernel,
    out_type=[jax.ShapeDtypeStruct((NP, B), jnp.float32),
              jax.ShapeDtypeStruct((NP, B), jnp.float32),
              jax.ShapeDtypeStruct((NP, B), jnp.float32)],
    mesh=_mesh,
    compiler_params=pltpu.CompilerParams(use_tc_tiling_on_sc=False),
    scratch_types=[
        pltpu.VMEM_SHARED((NP, B), jnp.float32),
        pltpu.VMEM_SHARED((NP, B), jnp.float32),
        pltpu.VMEM((EC, B), jnp.float32),
        pltpu.VMEM((EC, B), jnp.float32),
        pltpu.VMEM((4, 128), jnp.int32),
        pltpu.VMEM((4, 128), jnp.int32),
        pltpu.VMEM((16, 16), jnp.float32),
        pltpu.VMEM((16, 16), jnp.float32),
        pltpu.VMEM((2, 128), jnp.int32),
        pltpu.VMEM((2, 128), jnp.int32),
        pltpu.VMEM((NCH, B), jnp.float32),
        pltpu.VMEM((NCH, B), jnp.float32),
        pltpu.VMEM((NCH, B), jnp.float32),
        pltpu.VMEM((NCH, B), jnp.float32),
        pltpu.VMEM((NCH, B), jnp.float32),
        pltpu.VMEM((NCH // 16, 16), jnp.float32),
        pltpu.VMEM((NCH // 16, 16), jnp.float32),
        pltpu.VMEM((NCH, B), jnp.float32),
        pltpu.SemaphoreType.DMA,
        pltpu.SemaphoreType.DMA,
        pltpu.SemaphoreType.DMA,
        pltpu.SemaphoreType.DMA,
        pltpu.SemaphoreType.DMA,
        pltpu.SemaphoreType.DMA,
        pltpu.SemaphoreType.DMA,
    ])
def _rd_ode_sc(x0t, edata, wdata, kd_r, kr_r, *rest):
    _sc_body(x0t, edata, wdata, kd_r, kr_r, *rest)


@jax.jit
def kernel(inputs, ind, edge_index, edge_w, k_diff, k_reac):
    del ind
    x0 = inputs[:, 0, :, -1]
    x0t = jnp.zeros((NP, B), jnp.float32).at[:N].set(x0.T)

    pad_e = EP - E
    src = jnp.concatenate([edge_index[0].astype(jnp.int32),
                           jnp.zeros((pad_e,), jnp.int32)])
    dst = jnp.concatenate([edge_index[1].astype(jnp.int32),
                           jnp.full((pad_e,), NP - 1, jnp.int32)])
    w = jnp.concatenate([edge_w, jnp.zeros((pad_e,), jnp.float32)])
    nchunks = EP // EC
    edata = jnp.concatenate([src.reshape(nchunks, 2, 128),
                             dst.reshape(nchunks, 2, 128)],
                            axis=1).reshape(nchunks * 4, 128)
    wdata = w.reshape(nchunks * 16, 16)

    kd_r = jnp.zeros((NP,), jnp.float32).at[:N].set(k_diff).reshape(NP // 16, 16)
    kr_r = jnp.zeros((NP,), jnp.float32).at[:N].set(k_reac).reshape(NP // 16, 16)

    xb, _accb, _wrow = _rd_ode_sc(x0t, edata, wdata, kd_r, kr_r)
    return xb[:N].T[None]


# speedup vs baseline: 18.0206x; 1.1177x over previous
import functools
import jax
import jax.numpy as jnp
from jax import lax
from jax.experimental import pallas as pl
from jax.experimental.pallas import tpu as pltpu
from jax.experimental.pallas import tpu_sc as plsc

N = 50000
E = 1600000
B = 16
NSTEP = 4
DT = 1.0 / NSTEP
DT6 = DT / 6.0

NT = 16
NP = 50176
RANGE = NP // NT
NCH = 64
NCHN = RANGE // NCH

EC = 512
EP = 1605632
EPT = EP // NT
ECN = EPT // EC

_mesh = plsc.VectorSubcoreMesh(core_axis_name="c", subcore_axis_name="s",
                               num_cores=1)


def _sc_body(x0t, edata, wdata, kd_r, kr_r,
             xb, accb, wrow,
             xe_sh, agg_sh,
             rows0, rows1, ed0, ed1, wd0, wd1, si0, si1,
             agg0, xe0, wr0, acc0, xb0, kd0, kr0, z_b,
             sem_e0, sem_e1, sem_g0, sem_g1, sem_s0, sem_s1,
             sem_n0):
    sid = lax.axis_index("s")
    nbase = sid * RANGE
    cbase = sid * (RANGE // 16)
    cid0 = sid * ECN

    rows = (rows0, rows1)
    ed = (ed0, ed1)
    wd = (wd0, wd1)
    si = (si0, si1)
    agg_b, xe_b, wr_b, acc_b, xb_b, kd_b, kr_b = (
        agg0, xe0, wr0, acc0, xb0, kd0, kr0)
    sem_e = (sem_e0, sem_e1)
    sem_g = (sem_g0, sem_g1)
    sem_s = (sem_s0, sem_s1)
    sem_n = sem_n0

    def ed_fetch(q, p):
        return (pltpu.make_async_copy(edata.at[pl.ds((cid0 + q) * 8, 8)],
                                      ed[p], sem_e[p]),
                pltpu.make_async_copy(wdata.at[pl.ds((cid0 + q) * 32, 32)],
                                      wd[p], sem_e[p]))

    def gathers(p):
        return tuple(
            pltpu.make_async_copy(xe_sh.at[ed[p].at[r]],
                                  rows[p].at[pl.ds(r * 128, 128)], sem_g[p])
            for r in range(4))

    def scatters(p):
        return tuple(
            pltpu.make_async_copy(rows[p].at[pl.ds(r * 128, 128)],
                                  agg_sh.at[si[p].at[r]], sem_s[p])
            for r in range(4))

    def scale(p):
        r = rows[p]
        e = ed[p]
        s = si[p]
        for g in range(32):
            wv = wd[p][g]
            for l in range(16):
                j = g * 16 + l
                r[j] = r[j] * wv[l]
        for rr in range(4):
            for k in range(8):
                s[rr, pl.ds(k * 16, 16)] = e[4 + rr, pl.ds(k * 16, 16)]

    def edge_pass(carry):
        for d in ed_fetch(0, 0):
            d.start()
        for d in ed_fetch(0, 0):
            d.wait()
        for g in gathers(0):
            g.start()
        for d in ed_fetch(1, 1):
            d.start()

        def q2_body(q2, c):
            last = q2 >= ECN // 2 - 1
            for p in (0, 1):
                q = q2 * 2 + p
                if p == 0:
                    for d in ed_fetch(0, 1):
                        d.wait()
                else:
                    @pl.when(jnp.logical_not(last))
                    def _():
                        for d in ed_fetch(0, 0):
                            d.wait()
                for g in gathers(p):
                    g.wait()
                if p == 0:
                    @pl.when(q2 > 0)
                    def _():
                        for s in scatters(1):
                            s.wait()
                else:
                    for s in scatters(0):
                        s.wait()
                if p == 0:
                    for g in gathers(1):
                        g.start()
                else:
                    @pl.when(jnp.logical_not(last))
                    def _():
                        for g in gathers(0):
                            g.start()
                scale(p)
                for s in scatters(p):
                    s.start(add=True)
                @pl.when(jnp.logical_not(last))
                def _():
                    for d in ed_fetch(q + 2, p):
                        d.start()
            return c

        c = lax.fori_loop(0, ECN // 2, q2_body, carry)
        for s in scatters(1):
            s.wait()
        return c

    def node_io(m):
        nb = nbase + m * NCH
        cb = cbase + m * (NCH // 16)
        return (pltpu.make_async_copy(agg_sh.at[pl.ds(nb, NCH)],
                                      agg_b, sem_n),
                pltpu.make_async_copy(xe_sh.at[pl.ds(nb, NCH)],
                                      xe_b, sem_n),
                pltpu.make_async_copy(wrow.at[pl.ds(nb, NCH)],
                                      wr_b, sem_n),
                pltpu.make_async_copy(accb.at[pl.ds(nb, NCH)],
                                      acc_b, sem_n),
                pltpu.make_async_copy(xb.at[pl.ds(nb, NCH)],
                                      xb_b, sem_n),
                pltpu.make_async_copy(kd_r.at[pl.ds(cb, NCH // 16)],
                                      kd_b, sem_n),
                pltpu.make_async_copy(kr_r.at[pl.ds(cb, NCH // 16)],
                                      kr_b, sem_n))

    def node_compute(wgt, bz, ciz, az):
        for g in range(NCH // 16):
            kdv = kd_b[g]
            krv = kr_b[g]
            for l in range(16):
                j = g * 16 + l
                xe = xe_b[j]
                deriv = (agg_b[j] * kdv[l] - wr_b[j] * xe
                         - (xe * xe) * krv[l])
                acc_new = acc_b[j] + deriv * wgt
                t = acc_new * bz
                xb_new = xb_b[j] + t
                xe_b[j] = xb_new + deriv * ciz
                xb_b[j] = xb_new
                acc_b[j] = acc_new * az

    for j in range(NCH):
        z_b[j] = jnp.zeros((B,), jnp.float32) + 1.0
    def p_ones(m, c):
        pltpu.sync_copy(z_b, xe_sh.at[pl.ds(nbase + m * NCH, NCH)])
        return c
    lax.fori_loop(0, NCHN, p_ones, 0)
    for j in range(NCH):
        z_b[j] = jnp.zeros((B,), jnp.float32)
    def p_zero(m, c):
        nb = nbase + m * NCH
        pltpu.sync_copy(z_b, agg_sh.at[pl.ds(nb, NCH)])
        pltpu.sync_copy(z_b, accb.at[pl.ds(nb, NCH)])
        return c
    lax.fori_loop(0, NCHN, p_zero, 0)
    plsc.subcore_barrier()

    edge_pass(0)
    plsc.subcore_barrier()

    def p_w(m, c):
        nb = nbase + m * NCH
        cb = cbase + m * (NCH // 16)
        pltpu.sync_copy(agg_sh.at[pl.ds(nb, NCH)], agg_b)
        pltpu.sync_copy(kd_r.at[pl.ds(cb, NCH // 16)], kd_b)
        pltpu.sync_copy(kr_r.at[pl.ds(cb, NCH // 16)], kr_b)
        for g in range(NCH // 16):
            kdv = kd_b[g]
            krv = kr_b[g]
            for l in range(16):
                j = g * 16 + l
                wr_b[j] = agg_b[j] * kdv[l] - krv[l]
        pltpu.sync_copy(wr_b, wrow.at[pl.ds(nb, NCH)])
        pltpu.sync_copy(x0t.at[pl.ds(nb, NCH)], xe_b)
        pltpu.sync_copy(xe_b, xe_sh.at[pl.ds(nb, NCH)])
        pltpu.sync_copy(xe_b, xb.at[pl.ds(nb, NCH)])
        pltpu.sync_copy(z_b, agg_sh.at[pl.ds(nb, NCH)])
        return c
    lax.fori_loop(0, NCHN, p_w, 0)
    plsc.subcore_barrier()

    def eval_body(t, carry):
        i = lax.rem(t, 4)
        is3 = i == 3
        wgt = jnp.where((i == 1) | (i == 2), 2.0, 1.0).astype(jnp.float32)
        bz = jnp.where(is3, DT6, 0.0).astype(jnp.float32)
        ciz = jnp.where(is3, 0.0,
                        jnp.where(i == 2, DT, 0.5 * DT)).astype(jnp.float32)
        az = jnp.where(is3, 0.0, 1.0).astype(jnp.float32)

        edge_pass(carry)
        plsc.subcore_barrier()

        def node_chunk(m, c):
            nb = nbase + m * NCH
            cb = cbase + m * (NCH // 16)
            pltpu.sync_copy(agg_sh.at[pl.ds(nb, NCH)], agg_b)
            pltpu.sync_copy(xe_sh.at[pl.ds(nb, NCH)], xe_b)
            pltpu.sync_copy(wrow.at[pl.ds(nb, NCH)], wr_b)
            pltpu.sync_copy(accb.at[pl.ds(nb, NCH)], acc_b)
            pltpu.sync_copy(xb.at[pl.ds(nb, NCH)], xb_b)
            pltpu.sync_copy(kd_r.at[pl.ds(cb, NCH // 16)], kd_b)
            pltpu.sync_copy(kr_r.at[pl.ds(cb, NCH // 16)], kr_b)
            node_compute(wgt, bz, ciz, az)
            pltpu.sync_copy(xe_b, xe_sh.at[pl.ds(nb, NCH)])
            pltpu.sync_copy(acc_b, accb.at[pl.ds(nb, NCH)])
            pltpu.sync_copy(xb_b, xb.at[pl.ds(nb, NCH)])
            pltpu.sync_copy(z_b, agg_sh.at[pl.ds(nb, NCH)])
            return c
        lax.fori_loop(0, NCHN, node_chunk, carry)
        plsc.subcore_barrier()
        return carry

    lax.fori_loop(0, NSTEP * 4, eval_body, 0)


@functools.partial(
    pl.kernel,
    out_type=[jax.ShapeDtypeStruct((NP, B), jnp.float32),
              jax.ShapeDtypeStruct((NP, B), jnp.float32),
              jax.ShapeDtypeStruct((NP, B), jnp.float32)],
    mesh=_mesh,
    compiler_params=pltpu.CompilerParams(use_tc_tiling_on_sc=False),
    scratch_types=[
        pltpu.VMEM_SHARED((NP, B), jnp.float32),
        pltpu.VMEM_SHARED((NP, B), jnp.float32),
        pltpu.VMEM((EC, B), jnp.float32),
        pltpu.VMEM((EC, B), jnp.float32),
        pltpu.VMEM((8, 128), jnp.int32),
        pltpu.VMEM((8, 128), jnp.int32),
        pltpu.VMEM((32, 16), jnp.float32),
        pltpu.VMEM((32, 16), jnp.float32),
        pltpu.VMEM((4, 128), jnp.int32),
        pltpu.VMEM((4, 128), jnp.int32),
        pltpu.VMEM((NCH, B), jnp.float32),
        pltpu.VMEM((NCH, B), jnp.float32),
        pltpu.VMEM((NCH, B), jnp.float32),
        pltpu.VMEM((NCH, B), jnp.float32),
        pltpu.VMEM((NCH, B), jnp.float32),
        pltpu.VMEM((NCH // 16, 16), jnp.float32),
        pltpu.VMEM((NCH // 16, 16), jnp.float32),
        pltpu.VMEM((NCH, B), jnp.float32),
        pltpu.SemaphoreType.DMA,
        pltpu.SemaphoreType.DMA,
        pltpu.SemaphoreType.DMA,
        pltpu.SemaphoreType.DMA,
        pltpu.SemaphoreType.DMA,
        pltpu.SemaphoreType.DMA,
        pltpu.SemaphoreType.DMA,
    ])
def _rd_ode_sc(x0t, edata, wdata, kd_r, kr_r, *rest):
    _sc_body(x0t, edata, wdata, kd_r, kr_r, *rest)


@jax.jit
def kernel(inputs, ind, edge_index, edge_w, k_diff, k_reac):
    del ind
    x0 = inputs[:, 0, :, -1]
    x0t = jnp.zeros((NP, B), jnp.float32).at[:N].set(x0.T)

    pad_e = EP - E
    src = jnp.concatenate([edge_index[0].astype(jnp.int32),
                           jnp.zeros((pad_e,), jnp.int32)])
    dst = jnp.concatenate([edge_index[1].astype(jnp.int32),
                           jnp.full((pad_e,), NP - 1, jnp.int32)])
    w = jnp.concatenate([edge_w, jnp.zeros((pad_e,), jnp.float32)])
    nchunks = EP // EC
    edata = jnp.concatenate([src.reshape(nchunks, 4, 128),
                             dst.reshape(nchunks, 4, 128)],
                            axis=1).reshape(nchunks * 8, 128)
    wdata = w.reshape(nchunks * 32, 16)

    kd_r = jnp.zeros((NP,), jnp.float32).at[:N].set(k_diff).reshape(NP // 16, 16)
    kr_r = jnp.zeros((NP,), jnp.float32).at[:N].set(k_reac).reshape(NP // 16, 16)

    xb, _accb, _wrow = _rd_ode_sc(x0t, edata, wdata, kd_r, kr_r)
    return xb[:N].T[None]


# speedup vs baseline: 21.2066x; 1.1768x over previous
import functools
import jax
import jax.numpy as jnp
from jax import lax
from jax.experimental import pallas as pl
from jax.experimental.pallas import tpu as pltpu
from jax.experimental.pallas import tpu_sc as plsc

N = 50000
E = 1600000
B = 16
NSTEP = 4
DT = 1.0 / NSTEP
DT6 = DT / 6.0

NT = 16
NP = 50176
RANGE = NP // NT
NCH = 64
NCHN = RANGE // NCH

EC = 512
EP = 1605632
EPT = EP // NT
ECN = EPT // EC

_mesh = plsc.VectorSubcoreMesh(core_axis_name="c", subcore_axis_name="s",
                               num_cores=1)


def _sc_body(x0t, edata, wdata, kdkr,
             pstate,
             xe_sh, agg_sh,
             rows0, rows1, ed0, ed1, wd0, wd1, si0, si1,
             agg0, xe0, st_b, kk_b, z_b,
             sem_e0, sem_e1, sem_g0, sem_g1, sem_s0, sem_s1):
    sid = lax.axis_index("s")
    nbase = sid * RANGE
    cbase = sid * (RANGE // 16)
    cid0 = sid * ECN

    rows = (rows0, rows1)
    ed = (ed0, ed1)
    wd = (wd0, wd1)
    si = (si0, si1)
    agg_b, xe_b = agg0, xe0
    sem_e = (sem_e0, sem_e1)
    sem_g = (sem_g0, sem_g1)
    sem_s = (sem_s0, sem_s1)

    def ed_fetch(q, p):
        return (pltpu.make_async_copy(edata.at[pl.ds((cid0 + q) * 8, 8)],
                                      ed[p], sem_e[p]),
                pltpu.make_async_copy(wdata.at[pl.ds((cid0 + q) * 32, 32)],
                                      wd[p], sem_e[p]))

    def gathers(p):
        return tuple(
            pltpu.make_async_copy(xe_sh.at[ed[p].at[r]],
                                  rows[p].at[pl.ds(r * 128, 128)], sem_g[p])
            for r in range(4))

    def scatters(p):
        return tuple(
            pltpu.make_async_copy(rows[p].at[pl.ds(r * 128, 128)],
                                  agg_sh.at[si[p].at[r]], sem_s[p])
            for r in range(4))

    def scale(p):
        r = rows[p]
        e = ed[p]
        s = si[p]
        for g in range(32):
            wv = wd[p][g]
            for l in range(16):
                j = g * 16 + l
                r[j] = r[j] * wv[l]
        for rr in range(4):
            for k in range(8):
                s[rr, pl.ds(k * 16, 16)] = e[4 + rr, pl.ds(k * 16, 16)]

    def edge_pass(carry):
        for d in ed_fetch(0, 0):
            d.start()
        for d in ed_fetch(0, 0):
            d.wait()
        for g in gathers(0):
            g.start()
        for d in ed_fetch(1, 1):
            d.start()

        def q2_body(q2, c):
            last = q2 >= ECN // 2 - 1
            for p in (0, 1):
                q = q2 * 2 + p
                if p == 0:
                    for d in ed_fetch(0, 1):
                        d.wait()
                else:
                    @pl.when(jnp.logical_not(last))
                    def _():
                        for d in ed_fetch(0, 0):
                            d.wait()
                for g in gathers(p):
                    g.wait()
                if p == 0:
                    @pl.when(q2 > 0)
                    def _():
                        for s in scatters(1):
                            s.wait()
                else:
                    for s in scatters(0):
                        s.wait()
                if p == 0:
                    for g in gathers(1):
                        g.start()
                else:
                    @pl.when(jnp.logical_not(last))
                    def _():
                        for g in gathers(0):
                            g.start()
                scale(p)
                for s in scatters(p):
                    s.start(add=True)
                @pl.when(jnp.logical_not(last))
                def _():
                    for d in ed_fetch(q + 2, p):
                        d.start()
            return c

        c = lax.fori_loop(0, ECN // 2, q2_body, carry)
        for s in scatters(1):
            s.wait()
        return c

    def node_compute(wgt, bz, ciz, az):
        for g in range(NCH // 16):
            kdv = kk_b[g]
            krv = kk_b[NCH // 16 + g]
            for l in range(16):
                j = g * 16 + l
                xe = xe_b[j]
                deriv = (agg_b[j] * kdv[l] - st_b[j] * xe
                         - (xe * xe) * krv[l])
                acc_new = st_b[NCH + j] + deriv * wgt
                t = acc_new * bz
                xb_new = st_b[2 * NCH + j] + t
                xe_b[j] = xb_new + deriv * ciz
                st_b[2 * NCH + j] = xb_new
                st_b[NCH + j] = acc_new * az

    for j in range(NCH):
        z_b[j] = jnp.zeros((B,), jnp.float32) + 1.0
    def p_ones(m, c):
        pltpu.sync_copy(z_b, xe_sh.at[pl.ds(nbase + m * NCH, NCH)])
        return c
    lax.fori_loop(0, NCHN, p_ones, 0)
    for j in range(NCH):
        z_b[j] = jnp.zeros((B,), jnp.float32)
    def p_zero(m, c):
        nb = nbase + m * NCH
        pltpu.sync_copy(z_b, agg_sh.at[pl.ds(nb, NCH)])
        return c
    lax.fori_loop(0, NCHN, p_zero, 0)
    plsc.subcore_barrier()

    edge_pass(0)
    plsc.subcore_barrier()

    def p_w(m, c):
        nb = nbase + m * NCH
        gm = sid * NCHN + m
        b3 = gm * 3 * NCH
        pltpu.sync_copy(agg_sh.at[pl.ds(nb, NCH)], agg_b)
        pltpu.sync_copy(kdkr.at[pl.ds(gm * 2 * (NCH // 16), 2 * (NCH // 16))],
                        kk_b)
        for g in range(NCH // 16):
            kdv = kk_b[g]
            krv = kk_b[NCH // 16 + g]
            for l in range(16):
                j = g * 16 + l
                st_b[j] = agg_b[j] * kdv[l] - krv[l]
        pltpu.sync_copy(st_b.at[pl.ds(0, NCH)], pstate.at[pl.ds(b3, NCH)])
        pltpu.sync_copy(x0t.at[pl.ds(nb, NCH)], xe_b)
        pltpu.sync_copy(xe_b, xe_sh.at[pl.ds(nb, NCH)])
        pltpu.sync_copy(xe_b, pstate.at[pl.ds(b3 + 2 * NCH, NCH)])
        pltpu.sync_copy(z_b, pstate.at[pl.ds(b3 + NCH, NCH)])
        pltpu.sync_copy(z_b, agg_sh.at[pl.ds(nb, NCH)])
        return c
    lax.fori_loop(0, NCHN, p_w, 0)
    plsc.subcore_barrier()

    def eval_body(t, carry):
        i = lax.rem(t, 4)
        is3 = i == 3
        wgt = jnp.where((i == 1) | (i == 2), 2.0, 1.0).astype(jnp.float32)
        bz = jnp.where(is3, DT6, 0.0).astype(jnp.float32)
        ciz = jnp.where(is3, 0.0,
                        jnp.where(i == 2, DT, 0.5 * DT)).astype(jnp.float32)
        az = jnp.where(is3, 0.0, 1.0).astype(jnp.float32)

        edge_pass(carry)
        plsc.subcore_barrier()

        def node_chunk(m, c):
            nb = nbase + m * NCH
            gm = sid * NCHN + m
            b3 = gm * 3 * NCH
            pltpu.sync_copy(pstate.at[pl.ds(b3, 3 * NCH)], st_b)
            pltpu.sync_copy(agg_sh.at[pl.ds(nb, NCH)], agg_b)
            pltpu.sync_copy(xe_sh.at[pl.ds(nb, NCH)], xe_b)
            pltpu.sync_copy(
                kdkr.at[pl.ds(gm * 2 * (NCH // 16), 2 * (NCH // 16))], kk_b)
            node_compute(wgt, bz, ciz, az)
            pltpu.sync_copy(xe_b, xe_sh.at[pl.ds(nb, NCH)])
            pltpu.sync_copy(st_b.at[pl.ds(NCH, 2 * NCH)],
                            pstate.at[pl.ds(b3 + NCH, 2 * NCH)])
            pltpu.sync_copy(z_b, agg_sh.at[pl.ds(nb, NCH)])
            return c
        lax.fori_loop(0, NCHN, node_chunk, carry)
        plsc.subcore_barrier()
        return carry

    lax.fori_loop(0, NSTEP * 4, eval_body, 0)


@functools.partial(
    pl.kernel,
    out_type=jax.ShapeDtypeStruct((NP * 3, B), jnp.float32),
    mesh=_mesh,
    compiler_params=pltpu.CompilerParams(use_tc_tiling_on_sc=False),
    scratch_types=[
        pltpu.VMEM_SHARED((NP, B), jnp.float32),
        pltpu.VMEM_SHARED((NP, B), jnp.float32),
        pltpu.VMEM((EC, B), jnp.float32),
        pltpu.VMEM((EC, B), jnp.float32),
        pltpu.VMEM((8, 128), jnp.int32),
        pltpu.VMEM((8, 128), jnp.int32),
        pltpu.VMEM((32, 16), jnp.float32),
        pltpu.VMEM((32, 16), jnp.float32),
        pltpu.VMEM((4, 128), jnp.int32),
        pltpu.VMEM((4, 128), jnp.int32),
        pltpu.VMEM((NCH, B), jnp.float32),
        pltpu.VMEM((NCH, B), jnp.float32),
        pltpu.VMEM((3 * NCH, B), jnp.float32),
        pltpu.VMEM((2 * (NCH // 16), 16), jnp.float32),
        pltpu.VMEM((NCH, B), jnp.float32),
        pltpu.SemaphoreType.DMA,
        pltpu.SemaphoreType.DMA,
        pltpu.SemaphoreType.DMA,
        pltpu.SemaphoreType.DMA,
        pltpu.SemaphoreType.DMA,
        pltpu.SemaphoreType.DMA,
    ])
def _rd_ode_sc(x0t, edata, wdata, kdkr, *rest):
    _sc_body(x0t, edata, wdata, kdkr, *rest)


@jax.jit
def kernel(inputs, ind, edge_index, edge_w, k_diff, k_reac):
    del ind
    x0 = inputs[:, 0, :, -1]
    x0t = jnp.zeros((NP, B), jnp.float32).at[:N].set(x0.T)

    pad_e = EP - E
    src = jnp.concatenate([edge_index[0].astype(jnp.int32),
                           jnp.zeros((pad_e,), jnp.int32)])
    dst = jnp.concatenate([edge_index[1].astype(jnp.int32),
                           jnp.full((pad_e,), NP - 1, jnp.int32)])
    w = jnp.concatenate([edge_w, jnp.zeros((pad_e,), jnp.float32)])
    nchunks = EP // EC
    edata = jnp.concatenate([src.reshape(nchunks, 4, 128),
                             dst.reshape(nchunks, 4, 128)],
                            axis=1).reshape(nchunks * 8, 128)
    wdata = w.reshape(nchunks * 32, 16)

    npch = NP // NCH
    kd_3 = jnp.zeros((NP,), jnp.float32).at[:N].set(k_diff).reshape(
        npch, NCH // 16, 16)
    kr_3 = jnp.zeros((NP,), jnp.float32).at[:N].set(k_reac).reshape(
        npch, NCH // 16, 16)
    kdkr = jnp.concatenate([kd_3, kr_3], axis=1).reshape(
        npch * 2 * (NCH // 16), 16)

    ps = _rd_ode_sc(x0t, edata, wdata, kdkr)
    xbout = ps.reshape(npch, 3, NCH, B)[:, 2].reshape(NP, B)
    return xbout[:N].T[None]
